# Initial kernel scaffold; baseline (speedup 1.0000x reference)
import jax
import jax.numpy as jnp
from jax.experimental import pallas as pl


def kernel(x, node_idx, edge_idx, W0, b0, W1, b1, Wp0, bp0, Wp1, bp1):
    raise NotImplementedError("write your pallas kernel here")


# speedup vs baseline: 4.5314x; 4.5314x over previous
import functools

import jax
import jax.numpy as jnp
from jax import lax
from jax.experimental import pallas as pl
from jax.experimental.pallas import tpu as pltpu
from jax.experimental.pallas import tpu_sc as plsc

N = 10000
M = 5000
P = 320000
D = 128

NC, NS = 2, 16
NW = NC * NS
PPW = P // NW
C = 80
NCHUNK = PPW // C

MP = 5120
NP = 10240
MROWS = MP // NS
NROWS = NP // NS

F32 = jnp.float32


def _worker_id():
    return lax.axis_index("s") * NC + lax.axis_index("c")


@functools.cache
def _mesh():
    return plsc.VectorSubcoreMesh(
        core_axis_name="c", subcore_axis_name="s",
        num_cores=NC, num_subcores=NS)


@functools.cache
def _v2e_with_counts_kernel():
    return functools.partial(
        pl.kernel,
        mesh=_mesh(),
        out_type=(
            jax.ShapeDtypeStruct((NC * MP, D), F32),
            jax.ShapeDtypeStruct((NC * MP,), F32),
            jax.ShapeDtypeStruct((NC * NP,), F32),
        ),
        scratch_types=[
            pltpu.VMEM_SHARED((MP, D), F32),
            pltpu.VMEM_SHARED((MP,), F32),
            pltpu.VMEM_SHARED((NP,), F32),
            pltpu.VMEM((C,), jnp.int32),
            pltpu.VMEM((C,), jnp.int32),
            pltpu.VMEM((C, D), F32),
            pltpu.VMEM((C,), F32),
            pltpu.VMEM((NROWS,), F32),
            pltpu.SemaphoreType.DMA,
        ],
    )(_v2e_with_counts_body)


def _v2e_with_counts_body(tab_hbm, gi_hbm, si_hbm, zrow_hbm,
                          out_e, out_ec, out_vc,
                          acc, ecnt, vcnt, gidx, sidx, rows, ones_v, cnt_v,
                          sem):
    cid = lax.axis_index("c")
    sid = lax.axis_index("s")
    wid = _worker_id()
    pltpu.sync_copy(zrow_hbm.at[pl.ds(0, MROWS), :],
                    acc.at[pl.ds(sid * MROWS, MROWS), :])
    for i in range(NROWS // 16):
        cnt_v[pl.ds(i * 16, 16)] = jnp.zeros((16,), F32)
    pltpu.sync_copy(cnt_v.at[pl.ds(0, MROWS)],
                    ecnt.at[pl.ds(sid * MROWS, MROWS)])
    pltpu.sync_copy(cnt_v, vcnt.at[pl.ds(sid * NROWS, NROWS)])
    for i in range(C // 16):
        ones_v[pl.ds(i * 16, 16)] = jnp.ones((16,), F32)
    plsc.subcore_barrier()

    def body(j, carry):
        base = wid * PPW + j * C
        pltpu.sync_copy(gi_hbm.at[pl.ds(base, C)], gidx)
        pltpu.sync_copy(si_hbm.at[pl.ds(base, C)], sidx)
        pltpu.async_copy(tab_hbm.at[gidx], rows, sem).wait()
        pltpu.sync_copy(rows, acc.at[sidx], add=True)
        pltpu.sync_copy(ones_v, ecnt.at[sidx], add=True)
        pltpu.sync_copy(ones_v, vcnt.at[gidx], add=True)
        return carry

    lax.fori_loop(0, NCHUNK, body, 0)
    plsc.subcore_barrier()
    pltpu.sync_copy(acc.at[pl.ds(sid * MROWS, MROWS), :],
                    out_e.at[pl.ds(cid * MP + sid * MROWS, MROWS), :])
    pltpu.sync_copy(ecnt.at[pl.ds(sid * MROWS, MROWS)],
                    cnt_v.at[pl.ds(0, MROWS)])
    pltpu.sync_copy(cnt_v.at[pl.ds(0, MROWS)],
                    out_ec.at[pl.ds(cid * MP + sid * MROWS, MROWS)])
    pltpu.sync_copy(vcnt.at[pl.ds(sid * NROWS, NROWS)], cnt_v)
    pltpu.sync_copy(cnt_v, out_vc.at[pl.ds(cid * NP + sid * NROWS, NROWS)])


@functools.cache
def _make_seg(acc_rows):
    rows_per = acc_rows // NS

    @functools.partial(
        pl.kernel,
        mesh=_mesh(),
        out_type=jax.ShapeDtypeStruct((NC * acc_rows, D), F32),
        scratch_types=[
            pltpu.VMEM_SHARED((acc_rows, D), F32),
            pltpu.VMEM((C,), jnp.int32),
            pltpu.VMEM((C,), jnp.int32),
            pltpu.VMEM((C, D), F32),
            pltpu.SemaphoreType.DMA,
        ],
    )
    def seg(tab_hbm, gi_hbm, si_hbm, zrow_hbm, out, acc, gidx, sidx, rows, sem):
        cid = lax.axis_index("c")
        sid = lax.axis_index("s")
        wid = _worker_id()
        for r in range(rows_per // MROWS):
            pltpu.sync_copy(
                zrow_hbm.at[pl.ds(0, MROWS), :],
                acc.at[pl.ds(sid * rows_per + r * MROWS, MROWS), :])
        plsc.subcore_barrier()

        def body(j, carry):
            base = wid * PPW + j * C
            pltpu.sync_copy(gi_hbm.at[pl.ds(base, C)], gidx)
            pltpu.sync_copy(si_hbm.at[pl.ds(base, C)], sidx)
            pltpu.async_copy(tab_hbm.at[gidx], rows, sem).wait()
            pltpu.sync_copy(rows, acc.at[sidx], add=True)
            return carry

        lax.fori_loop(0, NCHUNK, body, 0)
        plsc.subcore_barrier()
        pltpu.sync_copy(acc.at[pl.ds(sid * rows_per, rows_per), :],
                        out.at[pl.ds(cid * acc_rows + sid * rows_per, rows_per), :])

    return seg


def _theta(x, w, b):
    def body(x_ref, w_ref, b_ref, o_ref):
        o_ref[...] = jnp.dot(x_ref[...], w_ref[...],
                             preferred_element_type=F32) + b_ref[...]
    return pl.pallas_call(
        body, out_shape=jax.ShapeDtypeStruct(x.shape, F32),
    )(x, w, b.reshape(1, D))


def _combine_first(ep0, ep1, ec0, ec1, vc0, vc1):
    def body(a_ref, b_ref, e0_ref, e1_ref, v0_ref, v1_ref,
             eo_ref, ie_ref, iv_ref):
        ie = 1.0 / jnp.maximum(e0_ref[...] + e1_ref[...], 1.0)
        iv = 1.0 / jnp.maximum(v0_ref[...] + v1_ref[...], 1.0)
        eo_ref[...] = (a_ref[...] + b_ref[...]) * ie
        ie_ref[...] = ie
        iv_ref[...] = iv
    return pl.pallas_call(
        body,
        out_shape=(
            jax.ShapeDtypeStruct((M, D), F32),
            jax.ShapeDtypeStruct((M, 1), F32),
            jax.ShapeDtypeStruct((N, 1), F32),
        ),
    )(ep0, ep1, ec0, ec1, vc0, vc1)


def _combine_scale(a, b, inv):
    def body(a_ref, b_ref, i_ref, o_ref):
        o_ref[...] = (a_ref[...] + b_ref[...]) * i_ref[...]
    return pl.pallas_call(
        body, out_shape=jax.ShapeDtypeStruct(a.shape, F32),
    )(a, b, inv)


def _combine_relu_theta(vp0, vp1, inv_v, w, b):
    def body(a_ref, b2_ref, i_ref, w_ref, bb_ref, o_ref):
        h = jnp.maximum((a_ref[...] + b2_ref[...]) * i_ref[...], 0.0)
        o_ref[...] = jnp.dot(h, w_ref[...],
                             preferred_element_type=F32) + bb_ref[...]
    return pl.pallas_call(
        body, out_shape=jax.ShapeDtypeStruct((N, D), F32),
    )(vp0, vp1, inv_v, w, b.reshape(1, D))


def _final_head(vp0, vp1, inv_v, wp0, bp0, wp1, bp1):
    def body(a_ref, b_ref, i_ref, w0_ref, b0_ref, w1_ref, b1_ref,
             z_ref, h_ref):
        h = (a_ref[...] + b_ref[...]) * i_ref[...]
        h_ref[...] = h
        t = jnp.maximum(jnp.dot(h, w0_ref[...],
                                preferred_element_type=F32) + b0_ref[...], 0.0)
        z_ref[...] = jnp.dot(t, w1_ref[...],
                             preferred_element_type=F32) + b1_ref[...]
    return pl.pallas_call(
        body,
        out_shape=(
            jax.ShapeDtypeStruct((N, D), F32),
            jax.ShapeDtypeStruct((N, D), F32),
        ),
    )(vp0, vp1, inv_v, wp0, bp0.reshape(1, D), wp1, bp1.reshape(1, D))


def kernel(x, node_idx, edge_idx, W0, b0, W1, b1, Wp0, bp0, Wp1, bp1):
    zrow = jnp.zeros((NROWS, D), F32)

    h0 = _theta(x, W0, b0)
    ep, ecp, vcp = _v2e_with_counts_kernel()(h0, node_idx, edge_idx, zrow)
    e0, inv_e, inv_v = _combine_first(
        ep[:M], ep[MP:MP + M],
        ecp[:M, None], ecp[MP:MP + M, None],
        vcp[:N, None], vcp[NP:NP + N, None])
    vp = _make_seg(NP)(e0, edge_idx, node_idx, zrow)

    t = _combine_relu_theta(vp[:N], vp[NP:NP + N], inv_v, W1, b1)
    ep2 = _make_seg(MP)(t, node_idx, edge_idx, zrow)
    e1 = _combine_scale(ep2[:M], ep2[MP:MP + M], inv_e)
    vp2 = _make_seg(NP)(e1, edge_idx, node_idx, zrow)

    z, h = _final_head(vp2[:N], vp2[NP:NP + N], inv_v, Wp0, bp0, Wp1, bp1)
    return (z, h)


# speedup vs baseline: 7.2932x; 1.6095x over previous
import functools

import jax
import jax.numpy as jnp
from jax import lax
from jax.experimental import pallas as pl
from jax.experimental.pallas import tpu as pltpu
from jax.experimental.pallas import tpu_sc as plsc

N = 10000
M = 5000
P = 320000
D = 128

NC, NS = 2, 16
NW = NC * NS
PPW = P // NW
C = 80
NCHUNK = PPW // C

MP = 5120
NP = 10240
MROWS = MP // NS
NROWS = NP // NS

F32 = jnp.float32

_WIN = [i * 16 for i in range(C // 16)] + ([C - 16] if C % 16 else [])


def _copy_row(src2d, chunk, dst1d):
    for s in _WIN:
        dst1d[pl.ds(s, 16)] = src2d[chunk, pl.ds(s, 16)]


@functools.cache
def _mesh():
    return plsc.VectorSubcoreMesh(
        core_axis_name="c", subcore_axis_name="s",
        num_cores=NC, num_subcores=NS)


@functools.cache
def _make_seg(acc_rows, with_counts=False):
    rows_per = acc_rows // NS
    nbuf = 1 if acc_rows > MP else 2

    out_types = [jax.ShapeDtypeStruct((NC * acc_rows, D), F32)]
    scratch = [
        pltpu.VMEM_SHARED((acc_rows, D), F32),
        pltpu.VMEM((NCHUNK, C), jnp.int32),
        pltpu.VMEM((NCHUNK, C), jnp.int32),
        pltpu.VMEM((C,), jnp.int32),
        pltpu.VMEM((C,), jnp.int32),
        pltpu.VMEM((C,), jnp.int32),
    ] + [pltpu.VMEM((C, D), F32) for _ in range(nbuf)] + [
        pltpu.SemaphoreType.DMA,
        pltpu.SemaphoreType.DMA,
    ]
    if with_counts:
        out_types += [jax.ShapeDtypeStruct((NC * MP,), F32),
                      jax.ShapeDtypeStruct((NC * NP,), F32)]
        scratch += [pltpu.VMEM_SHARED((MP,), F32),
                    pltpu.VMEM_SHARED((NP,), F32),
                    pltpu.VMEM((C,), F32),
                    pltpu.VMEM((NROWS,), F32)]

    def body_fn(*refs):
        if with_counts:
            (tab, gi2, si2, zrow, out, out_ec, out_vc,
             acc, gidx, sidx, gf0, gf1, sf, *rowbufs, sem0, sem1,
             ecnt, vcnt, ones_v, cnt_v) = refs
        else:
            (tab, gi2, si2, zrow, out,
             acc, gidx, sidx, gf0, gf1, sf, *rowbufs, sem0, sem1) = refs
        rows0 = rowbufs[0]
        rows1 = rowbufs[-1]
        cid = lax.axis_index("c")
        sid = lax.axis_index("s")
        wid = sid * NC + cid

        for r in range(rows_per // MROWS):
            pltpu.sync_copy(
                zrow.at[pl.ds(0, MROWS), :],
                acc.at[pl.ds(sid * rows_per + r * MROWS, MROWS), :])
        if with_counts:
            for i in range(NROWS // 16):
                cnt_v[pl.ds(i * 16, 16)] = jnp.zeros((16,), F32)
            pltpu.sync_copy(cnt_v.at[pl.ds(0, MROWS)],
                            ecnt.at[pl.ds(sid * MROWS, MROWS)])
            pltpu.sync_copy(cnt_v, vcnt.at[pl.ds(sid * NROWS, NROWS)])
            for i in range(C // 16):
                ones_v[pl.ds(i * 16, 16)] = jnp.ones((16,), F32)
        pltpu.sync_copy(gi2.at[wid], gidx)
        pltpu.sync_copy(si2.at[wid], sidx)
        plsc.subcore_barrier()

        def scatter(chunk, rows_buf, gflat):
            _copy_row(sidx, chunk, sf)
            pltpu.sync_copy(rows_buf, acc.at[sf], add=True)
            if with_counts:
                pltpu.sync_copy(ones_v, ecnt.at[sf], add=True)
                pltpu.sync_copy(ones_v, vcnt.at[gflat], add=True)

        if nbuf == 2:
            _copy_row(gidx, 0, gf0)
            pltpu.async_copy(tab.at[gf0], rows0, sem0)

            def loop_body(k, carry):
                a = 2 * k
                b = a + 1

                @pl.when(b < NCHUNK)
                def _():
                    _copy_row(gidx, b, gf1)

                pltpu.make_async_copy(tab.at[gf0], rows0, sem0).wait()

                @pl.when(b < NCHUNK)
                def _():
                    pltpu.async_copy(tab.at[gf1], rows1, sem1)

                scatter(a, rows0, gf0)

                @pl.when(b < NCHUNK)
                def _():
                    @pl.when(b + 1 < NCHUNK)
                    def _():
                        _copy_row(gidx, b + 1, gf0)

                    pltpu.make_async_copy(tab.at[gf1], rows1, sem1).wait()

                    @pl.when(b + 1 < NCHUNK)
                    def _():
                        pltpu.async_copy(tab.at[gf0], rows0, sem0)

                    scatter(b, rows1, gf1)

                return carry

            lax.fori_loop(0, (NCHUNK + 1) // 2, loop_body, 0)
        else:
            def loop_body(j, carry):
                _copy_row(gidx, j, gf0)
                pltpu.async_copy(tab.at[gf0], rows0, sem0).wait()
                scatter(j, rows0, gf0)
                return carry

            lax.fori_loop(0, NCHUNK, loop_body, 0)
        plsc.subcore_barrier()
        pltpu.sync_copy(
            acc.at[pl.ds(sid * rows_per, rows_per), :],
            out.at[pl.ds(cid * acc_rows + sid * rows_per, rows_per), :])
        if with_counts:
            pltpu.sync_copy(ecnt.at[pl.ds(sid * MROWS, MROWS)],
                            cnt_v.at[pl.ds(0, MROWS)])
            pltpu.sync_copy(cnt_v.at[pl.ds(0, MROWS)],
                            out_ec.at[pl.ds(cid * MP + sid * MROWS, MROWS)])
            pltpu.sync_copy(vcnt.at[pl.ds(sid * NROWS, NROWS)], cnt_v)
            pltpu.sync_copy(cnt_v,
                            out_vc.at[pl.ds(cid * NP + sid * NROWS, NROWS)])

    return functools.partial(
        pl.kernel,
        mesh=_mesh(),
        out_type=tuple(out_types) if with_counts else out_types[0],
        scratch_types=scratch,
    )(body_fn)


def _theta(x, w, b):
    def body(x_ref, w_ref, b_ref, o_ref):
        o_ref[...] = jnp.dot(x_ref[...], w_ref[...],
                             preferred_element_type=F32) + b_ref[...]
    return pl.pallas_call(
        body, out_shape=jax.ShapeDtypeStruct(x.shape, F32),
    )(x, w, b.reshape(1, D))


def _combine_first(ep0, ep1, ec0, ec1, vc0, vc1):
    def body(a_ref, b_ref, e0_ref, e1_ref, v0_ref, v1_ref,
             eo_ref, ie_ref, iv_ref):
        ie = 1.0 / jnp.maximum(e0_ref[...] + e1_ref[...], 1.0)
        iv = 1.0 / jnp.maximum(v0_ref[...] + v1_ref[...], 1.0)
        eo_ref[...] = (a_ref[...] + b_ref[...]) * ie
        ie_ref[...] = ie
        iv_ref[...] = iv
    return pl.pallas_call(
        body,
        out_shape=(
            jax.ShapeDtypeStruct((M, D), F32),
            jax.ShapeDtypeStruct((M, 1), F32),
            jax.ShapeDtypeStruct((N, 1), F32),
        ),
    )(ep0, ep1, ec0, ec1, vc0, vc1)


def _combine_scale(a, b, inv):
    def body(a_ref, b_ref, i_ref, o_ref):
        o_ref[...] = (a_ref[...] + b_ref[...]) * i_ref[...]
    return pl.pallas_call(
        body, out_shape=jax.ShapeDtypeStruct(a.shape, F32),
    )(a, b, inv)


def _combine_relu_theta(vp0, vp1, inv_v, w, b):
    def body(a_ref, b2_ref, i_ref, w_ref, bb_ref, o_ref):
        h = jnp.maximum((a_ref[...] + b2_ref[...]) * i_ref[...], 0.0)
        o_ref[...] = jnp.dot(h, w_ref[...],
                             preferred_element_type=F32) + bb_ref[...]
    return pl.pallas_call(
        body, out_shape=jax.ShapeDtypeStruct((N, D), F32),
    )(vp0, vp1, inv_v, w, b.reshape(1, D))


def _final_head(vp0, vp1, inv_v, wp0, bp0, wp1, bp1):
    def body(a_ref, b_ref, i_ref, w0_ref, b0_ref, w1_ref, b1_ref,
             z_ref, h_ref):
        h = (a_ref[...] + b_ref[...]) * i_ref[...]
        h_ref[...] = h
        t = jnp.maximum(jnp.dot(h, w0_ref[...],
                                preferred_element_type=F32) + b0_ref[...], 0.0)
        z_ref[...] = jnp.dot(t, w1_ref[...],
                             preferred_element_type=F32) + b1_ref[...]
    return pl.pallas_call(
        body,
        out_shape=(
            jax.ShapeDtypeStruct((N, D), F32),
            jax.ShapeDtypeStruct((N, D), F32),
        ),
    )(vp0, vp1, inv_v, wp0, bp0.reshape(1, D), wp1, bp1.reshape(1, D))


def kernel(x, node_idx, edge_idx, W0, b0, W1, b1, Wp0, bp0, Wp1, bp1):
    zrow = jnp.zeros((MROWS, D), F32)
    ni2 = node_idx.reshape(NW, NCHUNK, C)
    ei2 = edge_idx.reshape(NW, NCHUNK, C)

    h0 = _theta(x, W0, b0)
    ep, ecp, vcp = _make_seg(MP, True)(h0, ni2, ei2, zrow)
    e0, inv_e, inv_v = _combine_first(
        ep[:M], ep[MP:MP + M],
        ecp[:M, None], ecp[MP:MP + M, None],
        vcp[:N, None], vcp[NP:NP + N, None])
    vp = _make_seg(NP)(e0, ei2, ni2, zrow)

    t = _combine_relu_theta(vp[:N], vp[NP:NP + N], inv_v, W1, b1)
    ep2 = _make_seg(MP)(t, ni2, ei2, zrow)
    e1 = _combine_scale(ep2[:M], ep2[MP:MP + M], inv_e)
    vp2 = _make_seg(NP)(e1, ei2, ni2, zrow)

    z, h = _final_head(vp2[:N], vp2[NP:NP + N], inv_v, Wp0, bp0, Wp1, bp1)
    return (z, h)


# speedup vs baseline: 9.2065x; 1.2623x over previous
import functools

import jax
import jax.numpy as jnp
from jax import lax
from jax.experimental import pallas as pl
from jax.experimental.pallas import tpu as pltpu
from jax.experimental.pallas import tpu_sc as plsc

N = 10000
M = 5000
P = 320000
D = 128

NC, NS = 2, 16
NW = NC * NS
PPW = P // NW
C = 80
NCHUNK = PPW // C

MP = 5120
NP = 10240
MROWS = MP // NS
NROWS = NP // NS

F32 = jnp.float32

_WIN = [i * 16 for i in range(C // 16)] + ([C - 16] if C % 16 else [])


def _copy_row(src2d, chunk, dst1d):
    for s in _WIN:
        dst1d[pl.ds(s, 16)] = src2d[chunk, pl.ds(s, 16)]


@functools.cache
def _make_seg_np(acc_rows):
    assert NCHUNK % 2 == 1
    rows_per = acc_rows // NS

    @functools.partial(
        pl.kernel,
        mesh=_mesh(),
        out_type=jax.ShapeDtypeStruct((NC * acc_rows, D), F32),
        scratch_types=[
            pltpu.VMEM_SHARED((acc_rows, D), F32),
            pltpu.VMEM((2 * C,), jnp.int32),
            pltpu.VMEM((2 * C,), jnp.int32),
            pltpu.VMEM((C,), jnp.int32),
            pltpu.VMEM((C,), jnp.int32),
            pltpu.VMEM((C, D), F32),
            pltpu.VMEM((C, D), F32),
            pltpu.SemaphoreType.DMA,
            pltpu.SemaphoreType.DMA,
            pltpu.SemaphoreType.DMA,
            pltpu.SemaphoreType.DMA,
        ],
    )
    def seg(tab, iv, zrow, out,
            acc, buf0, buf1, sf0, sf1, rows0, rows1,
            semi0, semi1, semg0, semg1):
        cid = lax.axis_index("c")
        sid = lax.axis_index("s")
        wid = sid * NC + cid
        cbase = wid * NCHUNK

        for r in range(rows_per // MROWS):
            pltpu.sync_copy(
                zrow.at[pl.ds(0, MROWS), :],
                acc.at[pl.ds(sid * rows_per + r * MROWS, MROWS), :])
        plsc.subcore_barrier()

        def idx_start(j, buf, semi):
            pltpu.async_copy(iv.at[pl.ds((cbase + j) * 2 * C, 2 * C)],
                             buf, semi)

        def idx_wait(buf, semi):
            pltpu.make_async_copy(iv.at[pl.ds(0, 2 * C)], buf, semi).wait()

        def extract_sf(buf, sf):
            for s in _WIN:
                sf[pl.ds(s, 16)] = buf[pl.ds(C + s, 16)]

        def gather_start(buf, rows, semg):
            pltpu.async_copy(tab.at[buf.at[pl.ds(0, C)]], rows, semg)

        def gather_wait(buf, rows, semg):
            pltpu.make_async_copy(
                tab.at[buf.at[pl.ds(0, C)]], rows, semg).wait()

        idx_start(0, buf0, semi0)
        idx_start(1, buf1, semi1)
        idx_wait(buf0, semi0)
        gather_start(buf0, rows0, semg0)

        def pair(k, carry):
            j1 = 2 * k + 1
            idx_wait(buf1, semi1)
            gather_start(buf1, rows1, semg1)
            gather_wait(buf0, rows0, semg0)
            extract_sf(buf0, sf0)
            idx_start(j1 + 1, buf0, semi0)
            pltpu.sync_copy(rows0, acc.at[sf0], add=True)
            idx_wait(buf0, semi0)
            gather_start(buf0, rows0, semg0)
            gather_wait(buf1, rows1, semg1)
            extract_sf(buf1, sf1)

            @pl.when(j1 + 2 < NCHUNK)
            def _():
                idx_start(j1 + 2, buf1, semi1)

            pltpu.sync_copy(rows1, acc.at[sf1], add=True)
            return carry

        lax.fori_loop(0, NCHUNK // 2, pair, 0)
        gather_wait(buf0, rows0, semg0)
        extract_sf(buf0, sf0)
        pltpu.sync_copy(rows0, acc.at[sf0], add=True)

        plsc.subcore_barrier()
        pltpu.sync_copy(
            acc.at[pl.ds(sid * rows_per, rows_per), :],
            out.at[pl.ds(cid * acc_rows + sid * rows_per, rows_per), :])

    return seg


@functools.cache
def _mesh():
    return plsc.VectorSubcoreMesh(
        core_axis_name="c", subcore_axis_name="s",
        num_cores=NC, num_subcores=NS)


@functools.cache
def _make_seg(acc_rows, with_counts=False):
    rows_per = acc_rows // NS
    nbuf = 1 if acc_rows > MP else 2

    out_types = [jax.ShapeDtypeStruct((NC * acc_rows, D), F32)]
    scratch = [
        pltpu.VMEM_SHARED((acc_rows, D), F32),
        pltpu.VMEM((NCHUNK, C), jnp.int32),
        pltpu.VMEM((NCHUNK, C), jnp.int32),
        pltpu.VMEM((C,), jnp.int32),
        pltpu.VMEM((C,), jnp.int32),
        pltpu.VMEM((C,), jnp.int32),
    ] + [pltpu.VMEM((C, D), F32) for _ in range(nbuf)] + [
        pltpu.SemaphoreType.DMA,
        pltpu.SemaphoreType.DMA,
    ]
    if with_counts:
        out_types += [jax.ShapeDtypeStruct((NC * MP,), F32),
                      jax.ShapeDtypeStruct((NC * NP,), F32)]
        scratch += [pltpu.VMEM_SHARED((MP,), F32),
                    pltpu.VMEM_SHARED((NP,), F32),
                    pltpu.VMEM((C,), F32),
                    pltpu.VMEM((NROWS,), F32)]

    def body_fn(*refs):
        if with_counts:
            (tab, gi2, si2, zrow, out, out_ec, out_vc,
             acc, gidx, sidx, gf0, gf1, sf, *rowbufs, sem0, sem1,
             ecnt, vcnt, ones_v, cnt_v) = refs
        else:
            (tab, gi2, si2, zrow, out,
             acc, gidx, sidx, gf0, gf1, sf, *rowbufs, sem0, sem1) = refs
        rows0 = rowbufs[0]
        rows1 = rowbufs[-1]
        cid = lax.axis_index("c")
        sid = lax.axis_index("s")
        wid = sid * NC + cid

        for r in range(rows_per // MROWS):
            pltpu.sync_copy(
                zrow.at[pl.ds(0, MROWS), :],
                acc.at[pl.ds(sid * rows_per + r * MROWS, MROWS), :])
        if with_counts:
            for i in range(NROWS // 16):
                cnt_v[pl.ds(i * 16, 16)] = jnp.zeros((16,), F32)
            pltpu.sync_copy(cnt_v.at[pl.ds(0, MROWS)],
                            ecnt.at[pl.ds(sid * MROWS, MROWS)])
            pltpu.sync_copy(cnt_v, vcnt.at[pl.ds(sid * NROWS, NROWS)])
            for i in range(C // 16):
                ones_v[pl.ds(i * 16, 16)] = jnp.ones((16,), F32)
        pltpu.sync_copy(gi2.at[wid], gidx)
        pltpu.sync_copy(si2.at[wid], sidx)
        plsc.subcore_barrier()

        def scatter(chunk, rows_buf, gflat):
            _copy_row(sidx, chunk, sf)
            pltpu.sync_copy(rows_buf, acc.at[sf], add=True)
            if with_counts:
                pltpu.sync_copy(ones_v, ecnt.at[sf], add=True)
                pltpu.sync_copy(ones_v, vcnt.at[gflat], add=True)

        if nbuf == 2:
            _copy_row(gidx, 0, gf0)
            pltpu.async_copy(tab.at[gf0], rows0, sem0)

            def loop_body(k, carry):
                a = 2 * k
                b = a + 1

                @pl.when(b < NCHUNK)
                def _():
                    _copy_row(gidx, b, gf1)

                pltpu.make_async_copy(tab.at[gf0], rows0, sem0).wait()

                @pl.when(b < NCHUNK)
                def _():
                    pltpu.async_copy(tab.at[gf1], rows1, sem1)

                scatter(a, rows0, gf0)

                @pl.when(b < NCHUNK)
                def _():
                    @pl.when(b + 1 < NCHUNK)
                    def _():
                        _copy_row(gidx, b + 1, gf0)

                    pltpu.make_async_copy(tab.at[gf1], rows1, sem1).wait()

                    @pl.when(b + 1 < NCHUNK)
                    def _():
                        pltpu.async_copy(tab.at[gf0], rows0, sem0)

                    scatter(b, rows1, gf1)

                return carry

            lax.fori_loop(0, (NCHUNK + 1) // 2, loop_body, 0)
        else:
            def loop_body(j, carry):
                _copy_row(gidx, j, gf0)
                pltpu.async_copy(tab.at[gf0], rows0, sem0).wait()
                scatter(j, rows0, gf0)
                return carry

            lax.fori_loop(0, NCHUNK, loop_body, 0)
        plsc.subcore_barrier()
        pltpu.sync_copy(
            acc.at[pl.ds(sid * rows_per, rows_per), :],
            out.at[pl.ds(cid * acc_rows + sid * rows_per, rows_per), :])
        if with_counts:
            pltpu.sync_copy(ecnt.at[pl.ds(sid * MROWS, MROWS)],
                            cnt_v.at[pl.ds(0, MROWS)])
            pltpu.sync_copy(cnt_v.at[pl.ds(0, MROWS)],
                            out_ec.at[pl.ds(cid * MP + sid * MROWS, MROWS)])
            pltpu.sync_copy(vcnt.at[pl.ds(sid * NROWS, NROWS)], cnt_v)
            pltpu.sync_copy(cnt_v,
                            out_vc.at[pl.ds(cid * NP + sid * NROWS, NROWS)])

    return functools.partial(
        pl.kernel,
        mesh=_mesh(),
        out_type=tuple(out_types) if with_counts else out_types[0],
        scratch_types=scratch,
    )(body_fn)


def _theta(x, w, b):
    def body(x_ref, w_ref, b_ref, o_ref):
        o_ref[...] = jnp.dot(x_ref[...], w_ref[...],
                             preferred_element_type=F32) + b_ref[...]
    return pl.pallas_call(
        body, out_shape=jax.ShapeDtypeStruct(x.shape, F32),
    )(x, w, b.reshape(1, D))


def _combine_first(ep0, ep1, ec0, ec1, vc0, vc1):
    def body(a_ref, b_ref, e0_ref, e1_ref, v0_ref, v1_ref,
             eo_ref, ie_ref, iv_ref):
        ie = 1.0 / jnp.maximum(e0_ref[...] + e1_ref[...], 1.0)
        iv = 1.0 / jnp.maximum(v0_ref[...] + v1_ref[...], 1.0)
        eo_ref[...] = (a_ref[...] + b_ref[...]) * ie
        ie_ref[...] = ie
        iv_ref[...] = iv
    return pl.pallas_call(
        body,
        out_shape=(
            jax.ShapeDtypeStruct((M, D), F32),
            jax.ShapeDtypeStruct((M, 1), F32),
            jax.ShapeDtypeStruct((N, 1), F32),
        ),
    )(ep0, ep1, ec0, ec1, vc0, vc1)


def _combine_scale(a, b, inv):
    def body(a_ref, b_ref, i_ref, o_ref):
        o_ref[...] = (a_ref[...] + b_ref[...]) * i_ref[...]
    return pl.pallas_call(
        body, out_shape=jax.ShapeDtypeStruct(a.shape, F32),
    )(a, b, inv)


def _combine_relu_theta(vp0, vp1, inv_v, w, b):
    def body(a_ref, b2_ref, i_ref, w_ref, bb_ref, o_ref):
        h = jnp.maximum((a_ref[...] + b2_ref[...]) * i_ref[...], 0.0)
        o_ref[...] = jnp.dot(h, w_ref[...],
                             preferred_element_type=F32) + bb_ref[...]
    return pl.pallas_call(
        body, out_shape=jax.ShapeDtypeStruct((N, D), F32),
    )(vp0, vp1, inv_v, w, b.reshape(1, D))


def _final_head(vp0, vp1, inv_v, wp0, bp0, wp1, bp1):
    def body(a_ref, b_ref, i_ref, w0_ref, b0_ref, w1_ref, b1_ref,
             z_ref, h_ref):
        h = (a_ref[...] + b_ref[...]) * i_ref[...]
        h_ref[...] = h
        t = jnp.maximum(jnp.dot(h, w0_ref[...],
                                preferred_element_type=F32) + b0_ref[...], 0.0)
        z_ref[...] = jnp.dot(t, w1_ref[...],
                             preferred_element_type=F32) + b1_ref[...]
    return pl.pallas_call(
        body,
        out_shape=(
            jax.ShapeDtypeStruct((N, D), F32),
            jax.ShapeDtypeStruct((N, D), F32),
        ),
    )(vp0, vp1, inv_v, wp0, bp0.reshape(1, D), wp1, bp1.reshape(1, D))


def kernel(x, node_idx, edge_idx, W0, b0, W1, b1, Wp0, bp0, Wp1, bp1):
    zrow = jnp.zeros((MROWS, D), F32)
    ni2 = node_idx.reshape(NW, NCHUNK, C)
    ei2 = edge_idx.reshape(NW, NCHUNK, C)
    iv_e2v = jnp.stack(
        [edge_idx.reshape(-1, C), node_idx.reshape(-1, C)], axis=1).reshape(-1)

    h0 = _theta(x, W0, b0)
    ep, ecp, vcp = _make_seg(MP, True)(h0, ni2, ei2, zrow)
    e0, inv_e, inv_v = _combine_first(
        ep[:M], ep[MP:MP + M],
        ecp[:M, None], ecp[MP:MP + M, None],
        vcp[:N, None], vcp[NP:NP + N, None])
    vp = _make_seg_np(NP)(e0, iv_e2v, zrow)

    t = _combine_relu_theta(vp[:N], vp[NP:NP + N], inv_v, W1, b1)
    ep2 = _make_seg(MP)(t, ni2, ei2, zrow)
    e1 = _combine_scale(ep2[:M], ep2[MP:MP + M], inv_e)
    vp2 = _make_seg_np(NP)(e1, iv_e2v, zrow)

    z, h = _final_head(vp2[:N], vp2[NP:NP + N], inv_v, Wp0, bp0, Wp1, bp1)
    return (z, h)


# speedup vs baseline: 9.5056x; 1.0325x over previous
import functools

import jax
import jax.numpy as jnp
from jax import lax
from jax.experimental import pallas as pl
from jax.experimental.pallas import tpu as pltpu
from jax.experimental.pallas import tpu_sc as plsc

N = 10000
M = 5000
P = 320000
D = 128

NC, NS = 2, 16
NW = NC * NS
PPW = P // NW
C = 80
NCHUNK = PPW // C

MP = 5120
NP = 10240
MROWS = MP // NS
NROWS = NP // NS

F32 = jnp.float32

_WIN = [i * 16 for i in range(C // 16)]


@functools.cache
def _mesh():
    return plsc.VectorSubcoreMesh(
        core_axis_name="c", subcore_axis_name="s",
        num_cores=NC, num_subcores=NS)


@functools.cache
def _make_seg(acc_rows, with_counts=False):
    assert NCHUNK % 2 == 1
    rows_per = acc_rows // NS

    out_types = [jax.ShapeDtypeStruct((NC * acc_rows, D), F32)]
    scratch = [
        pltpu.VMEM_SHARED((acc_rows, D), F32),
        pltpu.VMEM((2 * C,), jnp.int32),
        pltpu.VMEM((2 * C,), jnp.int32),
        pltpu.VMEM((C,), jnp.int32),
        pltpu.VMEM((C,), jnp.int32),
        pltpu.VMEM((C, D), F32),
        pltpu.VMEM((C, D), F32),
        pltpu.SemaphoreType.DMA,
        pltpu.SemaphoreType.DMA,
        pltpu.SemaphoreType.DMA,
        pltpu.SemaphoreType.DMA,
    ]
    if with_counts:
        out_types += [jax.ShapeDtypeStruct((NC * MP,), F32),
                      jax.ShapeDtypeStruct((NC * NP,), F32)]
        scratch += [
            pltpu.VMEM_SHARED((MP,), F32),
            pltpu.VMEM_SHARED((NP,), F32),
            pltpu.VMEM((C,), jnp.int32),
            pltpu.VMEM((C,), jnp.int32),
            pltpu.VMEM((C,), F32),
            pltpu.VMEM((NROWS,), F32),
        ]

    def body_fn(*refs):
        if with_counts:
            (tab, iv, zrow, out, out_ec, out_vc,
             acc, buf0, buf1, sf0, sf1, rows0, rows1,
             semi0, semi1, semg0, semg1,
             ecnt, vcnt, gf0, gf1, ones_v, cnt_v) = refs
        else:
            (tab, iv, zrow, out,
             acc, buf0, buf1, sf0, sf1, rows0, rows1,
             semi0, semi1, semg0, semg1) = refs
            gf0 = gf1 = None
        cid = lax.axis_index("c")
        sid = lax.axis_index("s")
        wid = sid * NC + cid
        cbase = wid * NCHUNK

        for r in range(rows_per // MROWS):
            pltpu.sync_copy(
                zrow.at[pl.ds(0, MROWS), :],
                acc.at[pl.ds(sid * rows_per + r * MROWS, MROWS), :])
        if with_counts:
            for i in range(NROWS // 16):
                cnt_v[pl.ds(i * 16, 16)] = jnp.zeros((16,), F32)
            pltpu.sync_copy(cnt_v.at[pl.ds(0, MROWS)],
                            ecnt.at[pl.ds(sid * MROWS, MROWS)])
            pltpu.sync_copy(cnt_v, vcnt.at[pl.ds(sid * NROWS, NROWS)])
            for i in range(C // 16):
                ones_v[pl.ds(i * 16, 16)] = jnp.ones((16,), F32)
        plsc.subcore_barrier()

        def idx_start(j, buf, semi):
            pltpu.async_copy(iv.at[pl.ds((cbase + j) * 2 * C, 2 * C)],
                             buf, semi)

        def idx_wait(buf, semi):
            pltpu.make_async_copy(iv.at[pl.ds(0, 2 * C)], buf, semi).wait()

        def extract(buf, sf, gf):
            for s in _WIN:
                sf[pl.ds(s, 16)] = buf[pl.ds(C + s, 16)]
            if with_counts:
                for s in _WIN:
                    gf[pl.ds(s, 16)] = buf[pl.ds(s, 16)]

        def gather_start(buf, rows, semg):
            pltpu.async_copy(tab.at[buf.at[pl.ds(0, C)]], rows, semg)

        def gather_wait(buf, rows, semg):
            pltpu.make_async_copy(
                tab.at[buf.at[pl.ds(0, C)]], rows, semg).wait()

        def scatter(rows, sf, gf):
            pltpu.sync_copy(rows, acc.at[sf], add=True)
            if with_counts:
                pltpu.sync_copy(ones_v, ecnt.at[sf], add=True)
                pltpu.sync_copy(ones_v, vcnt.at[gf], add=True)

        idx_start(0, buf0, semi0)
        idx_start(1, buf1, semi1)
        idx_wait(buf0, semi0)
        gather_start(buf0, rows0, semg0)

        def pair(k, carry):
            j1 = 2 * k + 1
            idx_wait(buf1, semi1)
            gather_start(buf1, rows1, semg1)
            gather_wait(buf0, rows0, semg0)
            extract(buf0, sf0, gf0)
            idx_start(j1 + 1, buf0, semi0)
            scatter(rows0, sf0, gf0)
            idx_wait(buf0, semi0)
            gather_start(buf0, rows0, semg0)
            gather_wait(buf1, rows1, semg1)
            extract(buf1, sf1, gf1)

            @pl.when(j1 + 2 < NCHUNK)
            def _():
                idx_start(j1 + 2, buf1, semi1)

            scatter(rows1, sf1, gf1)
            return carry

        lax.fori_loop(0, NCHUNK // 2, pair, 0)
        gather_wait(buf0, rows0, semg0)
        extract(buf0, sf0, gf0)
        scatter(rows0, sf0, gf0)

        plsc.subcore_barrier()
        pltpu.sync_copy(
            acc.at[pl.ds(sid * rows_per, rows_per), :],
            out.at[pl.ds(cid * acc_rows + sid * rows_per, rows_per), :])
        if with_counts:
            pltpu.sync_copy(ecnt.at[pl.ds(sid * MROWS, MROWS)],
                            cnt_v.at[pl.ds(0, MROWS)])
            pltpu.sync_copy(cnt_v.at[pl.ds(0, MROWS)],
                            out_ec.at[pl.ds(cid * MP + sid * MROWS, MROWS)])
            pltpu.sync_copy(vcnt.at[pl.ds(sid * NROWS, NROWS)], cnt_v)
            pltpu.sync_copy(cnt_v,
                            out_vc.at[pl.ds(cid * NP + sid * NROWS, NROWS)])

    return functools.partial(
        pl.kernel,
        mesh=_mesh(),
        out_type=tuple(out_types) if with_counts else out_types[0],
        scratch_types=scratch,
    )(body_fn)


def _theta(x, w, b):
    def body(x_ref, w_ref, b_ref, o_ref):
        o_ref[...] = jnp.dot(x_ref[...], w_ref[...],
                             preferred_element_type=F32) + b_ref[...]
    return pl.pallas_call(
        body, out_shape=jax.ShapeDtypeStruct(x.shape, F32),
    )(x, w, b.reshape(1, D))


def _combine_first(ep0, ep1, ec0, ec1, vc0, vc1):
    def body(a_ref, b_ref, e0_ref, e1_ref, v0_ref, v1_ref,
             eo_ref, ie_ref, iv_ref):
        ie = 1.0 / jnp.maximum(e0_ref[...] + e1_ref[...], 1.0)
        iv = 1.0 / jnp.maximum(v0_ref[...] + v1_ref[...], 1.0)
        eo_ref[...] = (a_ref[...] + b_ref[...]) * ie
        ie_ref[...] = ie
        iv_ref[...] = iv
    return pl.pallas_call(
        body,
        out_shape=(
            jax.ShapeDtypeStruct((M, D), F32),
            jax.ShapeDtypeStruct((M, 1), F32),
            jax.ShapeDtypeStruct((N, 1), F32),
        ),
    )(ep0, ep1, ec0, ec1, vc0, vc1)


def _combine_scale(a, b, inv):
    def body(a_ref, b_ref, i_ref, o_ref):
        o_ref[...] = (a_ref[...] + b_ref[...]) * i_ref[...]
    return pl.pallas_call(
        body, out_shape=jax.ShapeDtypeStruct(a.shape, F32),
    )(a, b, inv)


def _combine_relu_theta(vp0, vp1, inv_v, w, b):
    def body(a_ref, b2_ref, i_ref, w_ref, bb_ref, o_ref):
        h = jnp.maximum((a_ref[...] + b2_ref[...]) * i_ref[...], 0.0)
        o_ref[...] = jnp.dot(h, w_ref[...],
                             preferred_element_type=F32) + bb_ref[...]
    return pl.pallas_call(
        body, out_shape=jax.ShapeDtypeStruct((N, D), F32),
    )(vp0, vp1, inv_v, w, b.reshape(1, D))


def _final_head(vp0, vp1, inv_v, wp0, bp0, wp1, bp1):
    def body(a_ref, b_ref, i_ref, w0_ref, b0_ref, w1_ref, b1_ref,
             z_ref, h_ref):
        h = (a_ref[...] + b_ref[...]) * i_ref[...]
        h_ref[...] = h
        t = jnp.maximum(jnp.dot(h, w0_ref[...],
                                preferred_element_type=F32) + b0_ref[...], 0.0)
        z_ref[...] = jnp.dot(t, w1_ref[...],
                             preferred_element_type=F32) + b1_ref[...]
    return pl.pallas_call(
        body,
        out_shape=(
            jax.ShapeDtypeStruct((N, D), F32),
            jax.ShapeDtypeStruct((N, D), F32),
        ),
    )(vp0, vp1, inv_v, wp0, bp0.reshape(1, D), wp1, bp1.reshape(1, D))


def kernel(x, node_idx, edge_idx, W0, b0, W1, b1, Wp0, bp0, Wp1, bp1):
    zrow = jnp.zeros((MROWS, D), F32)
    n2 = node_idx.reshape(-1, C)
    e2 = edge_idx.reshape(-1, C)
    iv_v2e = jnp.stack([n2, e2], axis=1).reshape(-1)
    iv_e2v = jnp.stack([e2, n2], axis=1).reshape(-1)

    h0 = _theta(x, W0, b0)
    ep, ecp, vcp = _make_seg(MP, True)(h0, iv_v2e, zrow)
    e0, inv_e, inv_v = _combine_first(
        ep[:M], ep[MP:MP + M],
        ecp[:M, None], ecp[MP:MP + M, None],
        vcp[:N, None], vcp[NP:NP + N, None])
    vp = _make_seg(NP)(e0, iv_e2v, zrow)

    t = _combine_relu_theta(vp[:N], vp[NP:NP + N], inv_v, W1, b1)
    ep2 = _make_seg(MP)(t, iv_v2e, zrow)
    e1 = _combine_scale(ep2[:M], ep2[MP:MP + M], inv_e)
    vp2 = _make_seg(NP)(e1, iv_e2v, zrow)

    z, h = _final_head(vp2[:N], vp2[NP:NP + N], inv_v, Wp0, bp0, Wp1, bp1)
    return (z, h)


# speedup vs baseline: 9.9721x; 1.0491x over previous
import functools

import jax
import jax.numpy as jnp
from jax import lax
from jax.experimental import pallas as pl
from jax.experimental.pallas import tpu as pltpu
from jax.experimental.pallas import tpu_sc as plsc

N = 10000
M = 5000
P = 320000
D = 128

NC, NS = 2, 16
NW = NC * NS
PPW = P // NW
C = 80
NCHUNK = PPW // C

MP = 5120
NP = 10240
MROWS = MP // NS
NROWS = NP // NS

F32 = jnp.float32

_WIN = [i * 16 for i in range(C // 16)]


@functools.cache
def _mesh():
    return plsc.VectorSubcoreMesh(
        core_axis_name="c", subcore_axis_name="s",
        num_cores=NC, num_subcores=NS)


@functools.cache
def _make_seg(acc_rows, with_counts=False):
    assert NCHUNK % 2 == 1
    rows_per = acc_rows // NS

    out_types = [jax.ShapeDtypeStruct((NC * acc_rows, D), F32)]
    scratch = [
        pltpu.VMEM_SHARED((acc_rows, D), F32),
        pltpu.VMEM((2 * C,), jnp.int32),
        pltpu.VMEM((2 * C,), jnp.int32),
        pltpu.VMEM((C,), jnp.int32),
        pltpu.VMEM((C,), jnp.int32),
        pltpu.VMEM((C, D), F32),
        pltpu.VMEM((C, D), F32),
        pltpu.SemaphoreType.DMA,
        pltpu.SemaphoreType.DMA,
        pltpu.SemaphoreType.DMA,
        pltpu.SemaphoreType.DMA,
    ]
    if with_counts:
        out_types += [jax.ShapeDtypeStruct((NC * MP,), F32),
                      jax.ShapeDtypeStruct((NC * NP,), F32)]
        scratch += [
            pltpu.VMEM_SHARED((MP,), F32),
            pltpu.VMEM_SHARED((NP,), F32),
            pltpu.VMEM((C,), jnp.int32),
            pltpu.VMEM((C,), jnp.int32),
            pltpu.VMEM((C,), F32),
            pltpu.VMEM((NROWS,), F32),
        ]

    def body_fn(*refs):
        if with_counts:
            (tab, iv, zrow, out, out_ec, out_vc,
             acc, buf0, buf1, sf0, sf1, rows0, rows1,
             semi0, semi1, semg0, semg1,
             ecnt, vcnt, gf0, gf1, ones_v, cnt_v) = refs
        else:
            (tab, iv, zrow, out,
             acc, buf0, buf1, sf0, sf1, rows0, rows1,
             semi0, semi1, semg0, semg1) = refs
            gf0 = gf1 = None
        cid = lax.axis_index("c")
        sid = lax.axis_index("s")
        wid = sid * NC + cid
        cbase = wid * NCHUNK

        for r in range(rows_per // MROWS):
            pltpu.sync_copy(
                zrow.at[pl.ds(0, MROWS), :],
                acc.at[pl.ds(sid * rows_per + r * MROWS, MROWS), :])
        if with_counts:
            for i in range(NROWS // 16):
                cnt_v[pl.ds(i * 16, 16)] = jnp.zeros((16,), F32)
            pltpu.sync_copy(cnt_v.at[pl.ds(0, MROWS)],
                            ecnt.at[pl.ds(sid * MROWS, MROWS)])
            pltpu.sync_copy(cnt_v, vcnt.at[pl.ds(sid * NROWS, NROWS)])
            for i in range(C // 16):
                ones_v[pl.ds(i * 16, 16)] = jnp.ones((16,), F32)
        plsc.subcore_barrier()

        def idx_start(j, buf, semi):
            pltpu.async_copy(iv.at[pl.ds((cbase + j) * 2 * C, 2 * C)],
                             buf, semi)

        def idx_wait(buf, semi):
            pltpu.make_async_copy(iv.at[pl.ds(0, 2 * C)], buf, semi).wait()

        def extract(buf, sf, gf):
            for s in _WIN:
                sf[pl.ds(s, 16)] = buf[pl.ds(C + s, 16)]
            if with_counts:
                for s in _WIN:
                    gf[pl.ds(s, 16)] = buf[pl.ds(s, 16)]

        def gather_start(buf, rows, semg):
            pltpu.async_copy(tab.at[buf.at[pl.ds(0, C)]], rows, semg)

        def gather_wait(buf, rows, semg):
            pltpu.make_async_copy(
                tab.at[buf.at[pl.ds(0, C)]], rows, semg).wait()

        def scatter(rows, sf, gf):
            pltpu.sync_copy(rows, acc.at[sf], add=True)
            if with_counts:
                pltpu.sync_copy(ones_v, ecnt.at[sf], add=True)
                pltpu.sync_copy(ones_v, vcnt.at[gf], add=True)

        idx_start(0, buf0, semi0)
        idx_start(1, buf1, semi1)
        idx_wait(buf0, semi0)
        gather_start(buf0, rows0, semg0)

        def pair(k, carry):
            j1 = 2 * k + 1
            idx_wait(buf1, semi1)
            gather_start(buf1, rows1, semg1)
            gather_wait(buf0, rows0, semg0)
            extract(buf0, sf0, gf0)
            idx_start(j1 + 1, buf0, semi0)
            scatter(rows0, sf0, gf0)
            idx_wait(buf0, semi0)
            gather_start(buf0, rows0, semg0)
            gather_wait(buf1, rows1, semg1)
            extract(buf1, sf1, gf1)

            @pl.when(j1 + 2 < NCHUNK)
            def _():
                idx_start(j1 + 2, buf1, semi1)

            scatter(rows1, sf1, gf1)
            return carry

        lax.fori_loop(0, NCHUNK // 2, pair, 0)
        gather_wait(buf0, rows0, semg0)
        extract(buf0, sf0, gf0)
        scatter(rows0, sf0, gf0)

        plsc.subcore_barrier()
        pltpu.sync_copy(
            acc.at[pl.ds(sid * rows_per, rows_per), :],
            out.at[pl.ds(cid * acc_rows + sid * rows_per, rows_per), :])
        if with_counts:
            pltpu.sync_copy(ecnt.at[pl.ds(sid * MROWS, MROWS)],
                            cnt_v.at[pl.ds(0, MROWS)])
            pltpu.sync_copy(cnt_v.at[pl.ds(0, MROWS)],
                            out_ec.at[pl.ds(cid * MP + sid * MROWS, MROWS)])
            pltpu.sync_copy(vcnt.at[pl.ds(sid * NROWS, NROWS)], cnt_v)
            pltpu.sync_copy(cnt_v,
                            out_vc.at[pl.ds(cid * NP + sid * NROWS, NROWS)])

    return functools.partial(
        pl.kernel,
        mesh=_mesh(),
        out_type=tuple(out_types) if with_counts else out_types[0],
        scratch_types=scratch,
    )(body_fn)


def _theta(x, w, b):
    def body(x_ref, w_ref, b_ref, o_ref):
        o_ref[...] = jnp.dot(x_ref[...], w_ref[...],
                             preferred_element_type=F32) + b_ref[...]
    return pl.pallas_call(
        body, out_shape=jax.ShapeDtypeStruct(x.shape, F32),
    )(x, w, b.reshape(1, D))


def _pair(ref, rows, pad_rows):
    return (ref[pl.ds(0, rows), :], ref[pl.ds(pad_rows, rows), :])


def _combine_first(ep, ec0, ec1, vc0, vc1):
    def body(ep_ref, e0_ref, e1_ref, v0_ref, v1_ref,
             eo_ref, ie_ref, iv_ref):
        ie = 1.0 / jnp.maximum(e0_ref[...] + e1_ref[...], 1.0)
        iv = 1.0 / jnp.maximum(v0_ref[...] + v1_ref[...], 1.0)
        a, b = _pair(ep_ref, M, MP)
        eo_ref[...] = (a + b) * ie
        ie_ref[...] = ie
        iv_ref[...] = iv
    return pl.pallas_call(
        body,
        out_shape=(
            jax.ShapeDtypeStruct((M, D), F32),
            jax.ShapeDtypeStruct((M, 1), F32),
            jax.ShapeDtypeStruct((N, 1), F32),
        ),
    )(ep, ec0, ec1, vc0, vc1)


def _combine_scale(ep, inv):
    def body(ep_ref, i_ref, o_ref):
        a, b = _pair(ep_ref, M, MP)
        o_ref[...] = (a + b) * i_ref[...]
    return pl.pallas_call(
        body, out_shape=jax.ShapeDtypeStruct((M, D), F32),
    )(ep, inv)


def _combine_relu_theta(vp, inv_v, w, b):
    def body(vp_ref, i_ref, w_ref, bb_ref, o_ref):
        a, b2 = _pair(vp_ref, N, NP)
        h = jnp.maximum((a + b2) * i_ref[...], 0.0)
        o_ref[...] = jnp.dot(h, w_ref[...],
                             preferred_element_type=F32) + bb_ref[...]
    return pl.pallas_call(
        body, out_shape=jax.ShapeDtypeStruct((N, D), F32),
    )(vp, inv_v, w, b.reshape(1, D))


def _final_head(vp, inv_v, wp0, bp0, wp1, bp1):
    def body(vp_ref, i_ref, w0_ref, b0_ref, w1_ref, b1_ref,
             z_ref, h_ref):
        a, b = _pair(vp_ref, N, NP)
        h = (a + b) * i_ref[...]
        h_ref[...] = h
        t = jnp.maximum(jnp.dot(h, w0_ref[...],
                                preferred_element_type=F32) + b0_ref[...], 0.0)
        z_ref[...] = jnp.dot(t, w1_ref[...],
                             preferred_element_type=F32) + b1_ref[...]
    return pl.pallas_call(
        body,
        out_shape=(
            jax.ShapeDtypeStruct((N, D), F32),
            jax.ShapeDtypeStruct((N, D), F32),
        ),
    )(vp, inv_v, wp0, bp0.reshape(1, D), wp1, bp1.reshape(1, D))


def kernel(x, node_idx, edge_idx, W0, b0, W1, b1, Wp0, bp0, Wp1, bp1):
    zrow = jnp.zeros((MROWS, D), F32)
    n2 = node_idx.reshape(-1, C)
    e2 = edge_idx.reshape(-1, C)
    iv_v2e = jnp.stack([n2, e2], axis=1).reshape(-1)
    iv_e2v = jnp.stack([e2, n2], axis=1).reshape(-1)

    h0 = _theta(x, W0, b0)
    ep, ecp, vcp = _make_seg(MP, True)(h0, iv_v2e, zrow)
    e0, inv_e, inv_v = _combine_first(
        ep,
        ecp[:M, None], ecp[MP:MP + M, None],
        vcp[:N, None], vcp[NP:NP + N, None])
    vp = _make_seg(NP)(e0, iv_e2v, zrow)

    t = _combine_relu_theta(vp, inv_v, W1, b1)
    ep2 = _make_seg(MP)(t, iv_v2e, zrow)
    e1 = _combine_scale(ep2, inv_e)
    vp2 = _make_seg(NP)(e1, iv_e2v, zrow)

    z, h = _final_head(vp2, inv_v, Wp0, bp0, Wp1, bp1)
    return (z, h)


# speedup vs baseline: 11.6411x; 1.1674x over previous
import functools

import jax
import jax.numpy as jnp
from jax import lax
from jax.experimental import pallas as pl
from jax.experimental.pallas import tpu as pltpu
from jax.experimental.pallas import tpu_sc as plsc

N = 10000
M = 5000
P = 320000
D = 128

NC, NS = 2, 16
NW = NC * NS
PPW = P // NW
C = 80
NCHUNK = PPW // C

MP = 5120
NP = 10240
MROWS = MP // NS
NROWS = NP // NS

F32 = jnp.float32

_WIN = [i * 16 for i in range(C // 16)]


@functools.cache
def _mesh():
    return plsc.VectorSubcoreMesh(
        core_axis_name="c", subcore_axis_name="s",
        num_cores=NC, num_subcores=NS)


@functools.cache
def _make_seg(acc_rows, with_counts=False):
    assert NCHUNK % 2 == 1
    rows_per = acc_rows // NS

    out_types = [jax.ShapeDtypeStruct((NC * acc_rows, D), F32)]
    scratch = (
        [pltpu.VMEM_SHARED((acc_rows, D), F32)]
        + [pltpu.VMEM((2 * C,), jnp.int32) for _ in range(3)]
        + [pltpu.VMEM((C,), jnp.int32) for _ in range(3)]
        + [pltpu.VMEM((C, D), F32) for _ in range(3)]
        + [pltpu.SemaphoreType.DMA for _ in range(9)]
    )
    if with_counts:
        out_types += [jax.ShapeDtypeStruct((NC * MP,), F32),
                      jax.ShapeDtypeStruct((NC * NP,), F32)]
        scratch += (
            [pltpu.VMEM_SHARED((MP,), F32),
             pltpu.VMEM_SHARED((NP,), F32)]
            + [pltpu.VMEM((C,), jnp.int32) for _ in range(3)]
            + [pltpu.VMEM((C,), F32),
               pltpu.VMEM((NROWS,), F32)]
        )

    def body_fn(*refs):
        if with_counts:
            (tab, iv, zrow, out, out_ec, out_vc, acc,
             b0, b1, b2, s0, s1, s2, r0, r1, r2,
             si0, si1, si2, sg0, sg1, sg2, ss0, ss1, ss2,
             ecnt, vcnt, g0, g1, g2, ones_v, cnt_v) = refs
            gf = [g0, g1, g2]
        else:
            (tab, iv, zrow, out, acc,
             b0, b1, b2, s0, s1, s2, r0, r1, r2,
             si0, si1, si2, sg0, sg1, sg2, ss0, ss1, ss2) = refs
            gf = [None, None, None]
        buf = [b0, b1, b2]
        sf = [s0, s1, s2]
        rows = [r0, r1, r2]
        semi = [si0, si1, si2]
        semg = [sg0, sg1, sg2]
        sems = [ss0, ss1, ss2]
        cid = lax.axis_index("c")
        sid = lax.axis_index("s")
        wid = sid * NC + cid
        cbase = wid * NCHUNK

        for r in range(rows_per // MROWS):
            pltpu.sync_copy(
                zrow.at[pl.ds(0, MROWS), :],
                acc.at[pl.ds(sid * rows_per + r * MROWS, MROWS), :])
        if with_counts:
            for i in range(NROWS // 16):
                cnt_v[pl.ds(i * 16, 16)] = jnp.zeros((16,), F32)
            pltpu.sync_copy(cnt_v.at[pl.ds(0, MROWS)],
                            ecnt.at[pl.ds(sid * MROWS, MROWS)])
            pltpu.sync_copy(cnt_v, vcnt.at[pl.ds(sid * NROWS, NROWS)])
            for i in range(C // 16):
                ones_v[pl.ds(i * 16, 16)] = jnp.ones((16,), F32)
        plsc.subcore_barrier()

        def idx_start(j, s):
            pltpu.async_copy(iv.at[pl.ds((cbase + j) * 2 * C, 2 * C)],
                             buf[s], semi[s])

        def idx_wait(s):
            pltpu.make_async_copy(iv.at[pl.ds(0, 2 * C)], buf[s],
                                  semi[s]).wait()

        def extract(s):
            for w in _WIN:
                sf[s][pl.ds(w, 16)] = buf[s][pl.ds(C + w, 16)]
            if with_counts:
                for w in _WIN:
                    gf[s][pl.ds(w, 16)] = buf[s][pl.ds(w, 16)]

        def gather_start(s):
            pltpu.async_copy(tab.at[buf[s].at[pl.ds(0, C)]], rows[s],
                             semg[s])

        def gather_wait(s):
            pltpu.make_async_copy(tab.at[buf[s].at[pl.ds(0, C)]], rows[s],
                                  semg[s]).wait()

        def scatter_start(s):
            pltpu.async_copy(rows[s], acc.at[sf[s]], sems[s], add=True)
            if with_counts:
                pltpu.sync_copy(ones_v, ecnt.at[sf[s]], add=True)
                pltpu.sync_copy(ones_v, vcnt.at[gf[s]], add=True)

        def scatter_wait(s):
            pltpu.make_async_copy(rows[s], acc.at[sf[s]], sems[s]).wait()

        idx_start(0, 0)
        idx_start(1, 1)
        idx_start(2, 2)
        idx_wait(0)
        gather_start(0)
        idx_wait(1)
        gather_start(1)
        gather_wait(0)
        extract(0)
        idx_start(3, 0)
        scatter_start(0)

        def ring(k, carry):
            idx_wait(2)

            @pl.when(k > 0)
            def _():
                scatter_wait(2)

            gather_start(2)
            gather_wait(1)
            extract(1)
            idx_start(3 * k + 4, 1)
            scatter_start(1)
            idx_wait(0)
            scatter_wait(0)
            gather_start(0)
            gather_wait(2)
            extract(2)

            @pl.when(k < (NCHUNK - 5) // 3)
            def _():
                idx_start(3 * k + 5, 2)

            scatter_start(2)
            idx_wait(1)
            scatter_wait(1)
            gather_start(1)
            gather_wait(0)
            extract(0)

            @pl.when(k < (NCHUNK - 5) // 3)
            def _():
                idx_start(3 * k + 6, 0)

            scatter_start(0)
            return carry

        lax.fori_loop(0, (NCHUNK - 2) // 3, ring, 0)
        gather_wait(1)
        extract(1)
        scatter_start(1)
        scatter_wait(0)
        scatter_wait(1)
        scatter_wait(2)

        plsc.subcore_barrier()
        pltpu.sync_copy(
            acc.at[pl.ds(sid * rows_per, rows_per), :],
            out.at[pl.ds(cid * acc_rows + sid * rows_per, rows_per), :])
        if with_counts:
            pltpu.sync_copy(ecnt.at[pl.ds(sid * MROWS, MROWS)],
                            cnt_v.at[pl.ds(0, MROWS)])
            pltpu.sync_copy(cnt_v.at[pl.ds(0, MROWS)],
                            out_ec.at[pl.ds(cid * MP + sid * MROWS, MROWS)])
            pltpu.sync_copy(vcnt.at[pl.ds(sid * NROWS, NROWS)], cnt_v)
            pltpu.sync_copy(cnt_v,
                            out_vc.at[pl.ds(cid * NP + sid * NROWS, NROWS)])

    return functools.partial(
        pl.kernel,
        mesh=_mesh(),
        out_type=tuple(out_types) if with_counts else out_types[0],
        scratch_types=scratch,
    )(body_fn)


def _theta(x, w, b):
    def body(x_ref, w_ref, b_ref, o_ref):
        o_ref[...] = jnp.dot(x_ref[...], w_ref[...],
                             preferred_element_type=F32) + b_ref[...]
    return pl.pallas_call(
        body, out_shape=jax.ShapeDtypeStruct(x.shape, F32),
    )(x, w, b.reshape(1, D))


def _pair(ref, rows, pad_rows):
    return (ref[pl.ds(0, rows), :], ref[pl.ds(pad_rows, rows), :])


def _combine_first(ep, ec0, ec1, vc0, vc1):
    def body(ep_ref, e0_ref, e1_ref, v0_ref, v1_ref,
             eo_ref, ie_ref, iv_ref):
        ie = 1.0 / jnp.maximum(e0_ref[...] + e1_ref[...], 1.0)
        iv = 1.0 / jnp.maximum(v0_ref[...] + v1_ref[...], 1.0)
        a, b = _pair(ep_ref, M, MP)
        eo_ref[...] = (a + b) * ie
        ie_ref[...] = ie
        iv_ref[...] = iv
    return pl.pallas_call(
        body,
        out_shape=(
            jax.ShapeDtypeStruct((M, D), F32),
            jax.ShapeDtypeStruct((M, 1), F32),
            jax.ShapeDtypeStruct((N, 1), F32),
        ),
    )(ep, ec0, ec1, vc0, vc1)


def _combine_scale(ep, inv):
    def body(ep_ref, i_ref, o_ref):
        a, b = _pair(ep_ref, M, MP)
        o_ref[...] = (a + b) * i_ref[...]
    return pl.pallas_call(
        body, out_shape=jax.ShapeDtypeStruct((M, D), F32),
    )(ep, inv)


def _combine_relu_theta(vp, inv_v, w, b):
    def body(vp_ref, i_ref, w_ref, bb_ref, o_ref):
        a, b2 = _pair(vp_ref, N, NP)
        h = jnp.maximum((a + b2) * i_ref[...], 0.0)
        o_ref[...] = jnp.dot(h, w_ref[...],
                             preferred_element_type=F32) + bb_ref[...]
    return pl.pallas_call(
        body, out_shape=jax.ShapeDtypeStruct((N, D), F32),
    )(vp, inv_v, w, b.reshape(1, D))


def _final_head(vp, inv_v, wp0, bp0, wp1, bp1):
    def body(vp_ref, i_ref, w0_ref, b0_ref, w1_ref, b1_ref,
             z_ref, h_ref):
        a, b = _pair(vp_ref, N, NP)
        h = (a + b) * i_ref[...]
        h_ref[...] = h
        t = jnp.maximum(jnp.dot(h, w0_ref[...],
                                preferred_element_type=F32) + b0_ref[...], 0.0)
        z_ref[...] = jnp.dot(t, w1_ref[...],
                             preferred_element_type=F32) + b1_ref[...]
    return pl.pallas_call(
        body,
        out_shape=(
            jax.ShapeDtypeStruct((N, D), F32),
            jax.ShapeDtypeStruct((N, D), F32),
        ),
    )(vp, inv_v, wp0, bp0.reshape(1, D), wp1, bp1.reshape(1, D))


def kernel(x, node_idx, edge_idx, W0, b0, W1, b1, Wp0, bp0, Wp1, bp1):
    zrow = jnp.zeros((MROWS, D), F32)
    n2 = node_idx.reshape(-1, C)
    e2 = edge_idx.reshape(-1, C)
    iv_v2e = jnp.stack([n2, e2], axis=1).reshape(-1)
    iv_e2v = jnp.stack([e2, n2], axis=1).reshape(-1)

    h0 = _theta(x, W0, b0)
    ep, ecp, vcp = _make_seg(MP, True)(h0, iv_v2e, zrow)
    e0, inv_e, inv_v = _combine_first(
        ep,
        ecp[:M, None], ecp[MP:MP + M, None],
        vcp[:N, None], vcp[NP:NP + N, None])
    vp = _make_seg(NP)(e0, iv_e2v, zrow)

    t = _combine_relu_theta(vp, inv_v, W1, b1)
    ep2 = _make_seg(MP)(t, iv_v2e, zrow)
    e1 = _combine_scale(ep2, inv_e)
    vp2 = _make_seg(NP)(e1, iv_e2v, zrow)

    z, h = _final_head(vp2, inv_v, Wp0, bp0, Wp1, bp1)
    return (z, h)
